# baseline (speedup 1.0000x reference)
import jax
import jax.numpy as jnp
from jax.experimental import pallas as pl
from jax.experimental.pallas import tpu as pltpu


def _fused_cnn_kernel(x_ref, T1_ref, t1m_ref, T2_ref, t2m_ref,
                      wf1_ref, t3_ref, wf2_ref, b2_ref, o_ref,
                      y1p_ref, ph_ref, y2p_ref):
    tb = x_ref.shape[0]
    f32 = jnp.float32

    bf16 = jnp.bfloat16
    xf = x_ref[...].reshape(tb * 32, 32)
    z32 = jnp.zeros((1, 32), bf16)
    u1 = jnp.concatenate([xf[1:], z32], axis=0)
    u2 = jnp.concatenate([xf[2:], z32, z32], axis=0)
    A1 = jnp.concatenate([xf, u1, u2], axis=1)
    B1 = jnp.dot(A1, T1_ref[...], preferred_element_type=f32)
    B1 = B1.reshape(tb, 32, 256)
    pw = jnp.maximum(B1[:, 0:28, 0:128], B1[:, 0:28, 128:256])
    y1p_ref[...] = jnp.maximum(pw + t1m_ref[...], 0.0).astype(bf16)

    ph_ref[:, 0, :] = jnp.zeros((tb, 128), bf16)
    ph_ref[:, 15, :] = jnp.zeros((tb, 128), bf16)
    for q in range(14):
        ph_ref[:, q + 1, :] = jnp.maximum(y1p_ref[:, 2 * q, :],
                                          y1p_ref[:, 2 * q + 1, :])

    phf = ph_ref[...].reshape(tb * 16, 128)
    z128 = jnp.zeros((1, 128), bf16)
    v1 = jnp.concatenate([phf[1:], z128], axis=0)
    v2 = jnp.concatenate([phf[2:], z128, z128], axis=0)
    A2 = jnp.concatenate([phf, v1, v2], axis=1)
    B2 = jnp.dot(A2, T2_ref[...], preferred_element_type=f32)
    B2 = B2.reshape(tb, 16, 1024)
    pw2 = jnp.maximum(B2[:, 0:14, 0:512], B2[:, 0:14, 512:1024])
    y2p_ref[...] = jnp.maximum(pw2 + t2m_ref[...], 0.0).astype(bf16)

    acc = jnp.zeros((tb, 128), f32)
    for h in range(7):
        slab = jnp.maximum(y2p_ref[:, 2 * h, :], y2p_ref[:, 2 * h + 1, :])
        acc = acc + jnp.dot(slab, wf1_ref[h], preferred_element_type=f32)

    h1 = jnp.maximum(acc + t3_ref[...], 0.0)
    out = jnp.dot(h1, wf2_ref[...], preferred_element_type=f32)
    o_ref[...] = out + b2_ref[...]


def _pick_tile(n, candidates):
    for c in candidates:
        if n % c == 0:
            return c
    return n


def _build_tables(w1, t1, w2, t2):
    f32 = jnp.float32
    kh, kw, c, p, jj = jnp.meshgrid(jnp.arange(3), jnp.arange(3),
                                    jnp.arange(6), jnp.arange(2),
                                    jnp.arange(14), indexing="ij")
    rows = kh * 32 + 2 * jj + p + kw
    cols = p * 128 + c * 16 + (jj + 1)
    vals = w1[kh * 3 + kw, c]
    T1 = jnp.zeros((96, 256), f32).at[rows, cols].set(vals)

    c1, p1, jj1 = jnp.meshgrid(jnp.arange(6), jnp.arange(2),
                               jnp.arange(14), indexing="ij")
    t1m = jnp.zeros((1, 256), f32).at[0, p1 * 128 + c1 * 16 + jj1 + 1].set(
        jnp.broadcast_to(t1[0, c1], c1.shape))

    kh2, kw2, c2, p2, jj2 = jnp.meshgrid(jnp.arange(3), jnp.arange(3),
                                         jnp.arange(6), jnp.arange(2),
                                         jnp.arange(7), indexing="ij")
    rows2 = (kh2 * 128 + c2 * 16 + 2 * jj2 + p2 + kw2)[..., None]
    cols2 = (p2 * 512 + jj2 * 64)[..., None] + jnp.arange(64)
    vals2 = w2[(kh2 * 3 + kw2) * 6 + c2, :]
    T2 = jnp.zeros((384, 1024), f32).at[
        jnp.broadcast_to(rows2, vals2.shape),
        jnp.broadcast_to(cols2, vals2.shape)].set(vals2)

    t2m = jnp.pad(jnp.tile(t2, (1, 7)), ((0, 0), (0, 64)))
    return T1, t1m[:, 0:128].reshape(1, 1, 128), T2, t2m.reshape(1, 1, 512)


def kernel(x_nchw, w1, t1, w2, t2, wf1, t3, wf2, b2):
    n = x_nchw.shape[0]
    x = x_nchw.reshape(n, 28, 28)
    xpad = jnp.pad(x, ((0, 0), (1, 3), (1, 3))).astype(jnp.bfloat16)
    T1, t1m, T2, t2m = _build_tables(w1, t1, w2, t2)
    T1 = T1.astype(jnp.bfloat16)
    T2 = T2.astype(jnp.bfloat16)
    wf1r = jnp.pad(wf1.reshape(7, 448, 128),
                   ((0, 0), (0, 64), (0, 0))).astype(jnp.bfloat16)

    tb = _pick_tile(n, (128, 64, 32, 16, 8, 4, 2, 1))
    return pl.pallas_call(
        _fused_cnn_kernel,
        out_shape=jax.ShapeDtypeStruct((n, 10), jnp.float32),
        grid=(n // tb,),
        in_specs=[
            pl.BlockSpec((tb, 32, 32), lambda i: (i, 0, 0)),
            pl.BlockSpec((96, 256), lambda i: (0, 0)),
            pl.BlockSpec((1, 1, 128), lambda i: (0, 0, 0)),
            pl.BlockSpec((384, 1024), lambda i: (0, 0)),
            pl.BlockSpec((1, 1, 512), lambda i: (0, 0, 0)),
            pl.BlockSpec((7, 512, 128), lambda i: (0, 0, 0)),
            pl.BlockSpec((1, 128), lambda i: (0, 0)),
            pl.BlockSpec((128, 10), lambda i: (0, 0)),
            pl.BlockSpec((1, 10), lambda i: (0, 0)),
        ],
        out_specs=pl.BlockSpec((tb, 10), lambda i: (i, 0)),
        scratch_shapes=[
            pltpu.VMEM((tb, 28, 128), jnp.bfloat16),
            pltpu.VMEM((tb, 16, 128), jnp.bfloat16),
            pltpu.VMEM((tb, 14, 512), jnp.bfloat16),
        ],
        compiler_params=pltpu.CompilerParams(
            dimension_semantics=("parallel",),
            vmem_limit_bytes=100 * 1024 * 1024,
        ),
    )(xpad, T1, t1m, T2, t2m, wf1r, t3, wf2, b2)


# speedup vs baseline: 1.3245x; 1.3245x over previous
import jax
import jax.numpy as jnp
from jax.experimental import pallas as pl
from jax.experimental.pallas import tpu as pltpu


def _fused_cnn_kernel(x_ref, T1_ref, t1m_ref, T2_ref, t2m_ref,
                      wf1_ref, t3_ref, wf2_ref, b2_ref, o_ref,
                      y1p_ref, ph_ref, y2p_ref):
    tb = x_ref.shape[0]
    f32 = jnp.float32

    xf = x_ref[...].reshape(tb * 32, 32)
    z32 = jnp.zeros((1, 32), f32)
    u1 = jnp.concatenate([xf[1:], z32], axis=0)
    u2 = jnp.concatenate([xf[2:], z32, z32], axis=0)
    A1 = jnp.concatenate([xf, u1, u2], axis=1)
    B1 = jnp.dot(A1, T1_ref[...], preferred_element_type=f32)
    B1 = B1.reshape(tb, 32, 256)
    pw = jnp.maximum(B1[:, 0:28, 0:128], B1[:, 0:28, 128:256])
    y1p_ref[...] = jnp.maximum(pw + t1m_ref[...], 0.0)

    ph_ref[:, 0, :] = jnp.zeros((tb, 128), f32)
    ph_ref[:, 15, :] = jnp.zeros((tb, 128), f32)
    for q in range(14):
        ph_ref[:, q + 1, :] = jnp.maximum(y1p_ref[:, 2 * q, :],
                                          y1p_ref[:, 2 * q + 1, :])

    phf = ph_ref[...].reshape(tb * 16, 128)
    z128 = jnp.zeros((1, 128), f32)
    v1 = jnp.concatenate([phf[1:], z128], axis=0)
    v2 = jnp.concatenate([phf[2:], z128, z128], axis=0)
    A2 = jnp.concatenate([phf, v1, v2], axis=1)
    B2 = jnp.dot(A2, T2_ref[...], preferred_element_type=f32)
    B2 = B2.reshape(tb, 16, 1024)
    pw2 = jnp.maximum(B2[:, 0:14, 0:512], B2[:, 0:14, 512:1024])
    y2p_ref[...] = jnp.maximum(pw2 + t2m_ref[...], 0.0)

    acc = jnp.zeros((tb, 128), f32)
    for h in range(7):
        slab = jnp.maximum(y2p_ref[:, 2 * h, :], y2p_ref[:, 2 * h + 1, :])
        acc = acc + jnp.dot(slab, wf1_ref[h], preferred_element_type=f32)

    h1 = jnp.maximum(acc + t3_ref[...], 0.0)
    out = jnp.dot(h1, wf2_ref[...], preferred_element_type=f32)
    o_ref[...] = out + b2_ref[...]


def _pick_tile(n, candidates):
    for c in candidates:
        if n % c == 0:
            return c
    return n


def _build_tables(w1, t1, w2, t2):
    f32 = jnp.float32
    kh, kw, c, p, jj = jnp.meshgrid(jnp.arange(3), jnp.arange(3),
                                    jnp.arange(6), jnp.arange(2),
                                    jnp.arange(14), indexing="ij")
    rows = kh * 32 + 2 * jj + p + kw
    cols = p * 128 + c * 16 + (jj + 1)
    vals = w1[kh * 3 + kw, c]
    T1 = jnp.zeros((96, 256), f32).at[rows, cols].set(vals)

    c1, p1, jj1 = jnp.meshgrid(jnp.arange(6), jnp.arange(2),
                               jnp.arange(14), indexing="ij")
    t1m = jnp.zeros((1, 256), f32).at[0, p1 * 128 + c1 * 16 + jj1 + 1].set(
        jnp.broadcast_to(t1[0, c1], c1.shape))

    kh2, kw2, c2, p2, jj2 = jnp.meshgrid(jnp.arange(3), jnp.arange(3),
                                         jnp.arange(6), jnp.arange(2),
                                         jnp.arange(7), indexing="ij")
    rows2 = (kh2 * 128 + c2 * 16 + 2 * jj2 + p2 + kw2)[..., None]
    cols2 = (p2 * 512 + jj2 * 64)[..., None] + jnp.arange(64)
    vals2 = w2[(kh2 * 3 + kw2) * 6 + c2, :]
    T2 = jnp.zeros((384, 1024), f32).at[
        jnp.broadcast_to(rows2, vals2.shape),
        jnp.broadcast_to(cols2, vals2.shape)].set(vals2)

    t2m = jnp.pad(jnp.tile(t2, (1, 7)), ((0, 0), (0, 64)))
    return T1, t1m[:, 0:128].reshape(1, 1, 128), T2, t2m.reshape(1, 1, 512)


def kernel(x_nchw, w1, t1, w2, t2, wf1, t3, wf2, b2):
    n = x_nchw.shape[0]
    x = x_nchw.reshape(n, 28, 28)
    xpad = jnp.pad(x, ((0, 0), (1, 3), (1, 3)))
    T1, t1m, T2, t2m = _build_tables(w1, t1, w2, t2)
    wf1r = jnp.pad(wf1.reshape(7, 448, 128), ((0, 0), (0, 64), (0, 0)))

    tb = _pick_tile(n, (128, 64, 32, 16, 8, 4, 2, 1))
    return pl.pallas_call(
        _fused_cnn_kernel,
        out_shape=jax.ShapeDtypeStruct((n, 10), jnp.float32),
        grid=(n // tb,),
        in_specs=[
            pl.BlockSpec((tb, 32, 32), lambda i: (i, 0, 0)),
            pl.BlockSpec((96, 256), lambda i: (0, 0)),
            pl.BlockSpec((1, 1, 128), lambda i: (0, 0, 0)),
            pl.BlockSpec((384, 1024), lambda i: (0, 0)),
            pl.BlockSpec((1, 1, 512), lambda i: (0, 0, 0)),
            pl.BlockSpec((7, 512, 128), lambda i: (0, 0, 0)),
            pl.BlockSpec((1, 128), lambda i: (0, 0)),
            pl.BlockSpec((128, 10), lambda i: (0, 0)),
            pl.BlockSpec((1, 10), lambda i: (0, 0)),
        ],
        out_specs=pl.BlockSpec((tb, 10), lambda i: (i, 0)),
        scratch_shapes=[
            pltpu.VMEM((tb, 28, 128), jnp.float32),
            pltpu.VMEM((tb, 16, 128), jnp.float32),
            pltpu.VMEM((tb, 14, 512), jnp.float32),
        ],
        compiler_params=pltpu.CompilerParams(
            dimension_semantics=("arbitrary",),
            vmem_limit_bytes=100 * 1024 * 1024,
        ),
    )(xpad, T1, t1m, T2, t2m, wf1r, t3, wf2, b2)


# speedup vs baseline: 1.4988x; 1.1316x over previous
import jax
import jax.numpy as jnp
from jax.experimental import pallas as pl
from jax.experimental.pallas import tpu as pltpu


def _fused_cnn_kernel(x_ref, T1_ref, t1m_ref, T2_ref, t2m_ref,
                      wf1_ref, t3_ref, wf2_ref, b2_ref, o_ref,
                      xs_ref, y1p_ref, ph_ref, y2p_ref):
    tb = x_ref.shape[0]
    f32 = jnp.float32

    xs_ref[:, 0, :] = jnp.zeros((tb, 28), f32)
    xs_ref[:, 29:32, :] = jnp.zeros((tb, 3, 28), f32)
    xs_ref[:, 1:29, :] = x_ref[...]

    xf = xs_ref[...].reshape(tb * 32, 28)
    z32 = jnp.zeros((1, 28), f32)
    u1 = jnp.concatenate([xf[1:], z32], axis=0)
    u2 = jnp.concatenate([xf[2:], z32, z32], axis=0)
    A1 = jnp.concatenate([xf, u1, u2], axis=1)
    B1 = jnp.dot(A1, T1_ref[...], preferred_element_type=f32)
    B1 = B1.reshape(tb, 32, 256)
    pw = jnp.maximum(B1[:, 0:28, 0:128], B1[:, 0:28, 128:256])
    y1p_ref[...] = jnp.maximum(pw + t1m_ref[...], 0.0)

    ph_ref[:, 0, :] = jnp.zeros((tb, 128), f32)
    ph_ref[:, 15, :] = jnp.zeros((tb, 128), f32)
    for q in range(14):
        ph_ref[:, q + 1, :] = jnp.maximum(y1p_ref[:, 2 * q, :],
                                          y1p_ref[:, 2 * q + 1, :])

    phf = ph_ref[...].reshape(tb * 16, 128)
    z128 = jnp.zeros((1, 128), f32)
    v1 = jnp.concatenate([phf[1:], z128], axis=0)
    v2 = jnp.concatenate([phf[2:], z128, z128], axis=0)
    A2 = jnp.concatenate([phf, v1, v2], axis=1)
    B2 = jnp.dot(A2, T2_ref[...], preferred_element_type=f32)
    B2 = B2.reshape(tb, 16, 1024)
    pw2 = jnp.maximum(B2[:, 0:14, 0:512], B2[:, 0:14, 512:1024])
    y2p_ref[...] = jnp.maximum(pw2 + t2m_ref[...], 0.0)

    acc = jnp.zeros((tb, 128), f32)
    for h in range(7):
        slab = jnp.maximum(y2p_ref[:, 2 * h, :], y2p_ref[:, 2 * h + 1, :])
        acc = acc + jnp.dot(slab, wf1_ref[h], preferred_element_type=f32)

    h1 = jnp.maximum(acc + t3_ref[...], 0.0)
    out = jnp.dot(h1, wf2_ref[...], preferred_element_type=f32)
    o_ref[...] = out + b2_ref[...]


def _pick_tile(n, candidates):
    for c in candidates:
        if n % c == 0:
            return c
    return n


def _build_tables(w1, t1, w2, t2):
    f32 = jnp.float32
    kh, kw, c, p, jj = jnp.meshgrid(jnp.arange(3), jnp.arange(3),
                                    jnp.arange(6), jnp.arange(2),
                                    jnp.arange(14), indexing="ij")
    scol = 2 * jj + p + kw - 1
    valid = (scol >= 0) & (scol <= 27)
    rows = kh * 28 + jnp.clip(scol, 0, 27)
    cols = p * 128 + c * 16 + (jj + 1)
    vals = w1[kh * 3 + kw, c] * valid.astype(f32)
    T1 = jnp.zeros((84, 256), f32).at[rows, cols].add(vals)

    c1, p1, jj1 = jnp.meshgrid(jnp.arange(6), jnp.arange(2),
                               jnp.arange(14), indexing="ij")
    t1m = jnp.zeros((1, 256), f32).at[0, p1 * 128 + c1 * 16 + jj1 + 1].set(
        jnp.broadcast_to(t1[0, c1], c1.shape))

    kh2, kw2, c2, p2, jj2 = jnp.meshgrid(jnp.arange(3), jnp.arange(3),
                                         jnp.arange(6), jnp.arange(2),
                                         jnp.arange(7), indexing="ij")
    rows2 = (kh2 * 128 + c2 * 16 + 2 * jj2 + p2 + kw2)[..., None]
    cols2 = (p2 * 512 + jj2 * 64)[..., None] + jnp.arange(64)
    vals2 = w2[(kh2 * 3 + kw2) * 6 + c2, :]
    T2 = jnp.zeros((384, 1024), f32).at[
        jnp.broadcast_to(rows2, vals2.shape),
        jnp.broadcast_to(cols2, vals2.shape)].set(vals2)

    t2m = jnp.pad(jnp.tile(t2, (1, 7)), ((0, 0), (0, 64)))
    return T1, t1m[:, 0:128].reshape(1, 1, 128), T2, t2m.reshape(1, 1, 512)


def kernel(x_nchw, w1, t1, w2, t2, wf1, t3, wf2, b2):
    n = x_nchw.shape[0]
    x = x_nchw.reshape(n, 28, 28)
    T1, t1m, T2, t2m = _build_tables(w1, t1, w2, t2)
    wf1r = jnp.pad(wf1.reshape(7, 448, 128), ((0, 0), (0, 64), (0, 0)))

    tb = _pick_tile(n, (128, 64, 32, 16, 8, 4, 2, 1))
    return pl.pallas_call(
        _fused_cnn_kernel,
        out_shape=jax.ShapeDtypeStruct((n, 10), jnp.float32),
        grid=(n // tb,),
        in_specs=[
            pl.BlockSpec((tb, 28, 28), lambda i: (i, 0, 0)),
            pl.BlockSpec((84, 256), lambda i: (0, 0)),
            pl.BlockSpec((1, 1, 128), lambda i: (0, 0, 0)),
            pl.BlockSpec((384, 1024), lambda i: (0, 0)),
            pl.BlockSpec((1, 1, 512), lambda i: (0, 0, 0)),
            pl.BlockSpec((7, 512, 128), lambda i: (0, 0, 0)),
            pl.BlockSpec((1, 128), lambda i: (0, 0)),
            pl.BlockSpec((128, 10), lambda i: (0, 0)),
            pl.BlockSpec((1, 10), lambda i: (0, 0)),
        ],
        out_specs=pl.BlockSpec((tb, 10), lambda i: (i, 0)),
        scratch_shapes=[
            pltpu.VMEM((tb, 32, 28), jnp.float32),
            pltpu.VMEM((tb, 28, 128), jnp.float32),
            pltpu.VMEM((tb, 16, 128), jnp.float32),
            pltpu.VMEM((tb, 14, 512), jnp.float32),
        ],
        compiler_params=pltpu.CompilerParams(
            dimension_semantics=("parallel",),
            vmem_limit_bytes=100 * 1024 * 1024,
        ),
    )(x, T1, t1m, T2, t2m, wf1r, t3, wf2, b2)


# speedup vs baseline: 1.5390x; 1.0268x over previous
import jax
import jax.numpy as jnp
from jax.experimental import pallas as pl
from jax.experimental.pallas import tpu as pltpu


def _fused_cnn_kernel(x_ref, T1_ref, t1m_ref, T2_ref, t2m_ref,
                      wf1_ref, t3_ref, wf2_ref, b2_ref, o_ref,
                      xs_ref, y1p_ref, ph_ref, y2p_ref):
    tb = x_ref.shape[0]
    f32 = jnp.float32

    xs_ref[:, 0, :] = jnp.zeros((tb, 28), f32)
    xs_ref[:, 29:32, :] = jnp.zeros((tb, 3, 28), f32)
    xs_ref[:, 1:29, :] = x_ref[...]

    xf = xs_ref[...].reshape(tb * 32, 28)
    z32 = jnp.zeros((1, 28), f32)
    u1 = jnp.concatenate([xf[1:], z32], axis=0)
    u2 = jnp.concatenate([xf[2:], z32, z32], axis=0)
    A1 = jnp.concatenate([xf, u1, u2], axis=1)
    B1 = jnp.dot(A1, T1_ref[...], preferred_element_type=f32)
    B1 = B1.reshape(tb, 32, 256)
    pw = jnp.maximum(B1[:, 0:28, 0:128], B1[:, 0:28, 128:256])
    y1p_ref[...] = jnp.maximum(pw + t1m_ref[...], 0.0)

    ph_ref[:, 0, :] = jnp.zeros((tb, 128), f32)
    ph_ref[:, 15, :] = jnp.zeros((tb, 128), f32)
    for q in range(14):
        ph_ref[:, q + 1, :] = jnp.maximum(y1p_ref[:, 2 * q, :],
                                          y1p_ref[:, 2 * q + 1, :])

    phf = ph_ref[...].reshape(tb * 16, 128)
    z128 = jnp.zeros((1, 128), f32)
    v1 = jnp.concatenate([phf[1:], z128], axis=0)
    v2 = jnp.concatenate([phf[2:], z128, z128], axis=0)
    A2 = jnp.concatenate([phf, v1, v2], axis=1)
    B2 = jnp.dot(A2, T2_ref[...], preferred_element_type=f32)
    B2 = B2.reshape(tb, 16, 1024)
    pw2 = jnp.maximum(B2[:, 0:14, 0:512], B2[:, 0:14, 512:1024])
    y2p_ref[...] = jnp.maximum(pw2 + t2m_ref[...], 0.0)

    acc = jnp.zeros((tb, 128), f32)
    for h in range(7):
        slab = jnp.maximum(y2p_ref[:, 2 * h, :], y2p_ref[:, 2 * h + 1, :])
        acc = acc + jnp.dot(slab, wf1_ref[h], preferred_element_type=f32)

    h1 = jnp.maximum(acc + t3_ref[...], 0.0)
    out = jnp.dot(h1, wf2_ref[...], preferred_element_type=f32)
    o_ref[...] = out + b2_ref[...]


def _pick_tile(n, candidates):
    for c in candidates:
        if n % c == 0:
            return c
    return n


def _build_tables(w1, t1, w2, t2):
    f32 = jnp.float32
    kh, kw, c, p, jj = jnp.meshgrid(jnp.arange(3), jnp.arange(3),
                                    jnp.arange(6), jnp.arange(2),
                                    jnp.arange(14), indexing="ij")
    scol = 2 * jj + p + kw - 1
    valid = (scol >= 0) & (scol <= 27)
    rows = kh * 28 + jnp.clip(scol, 0, 27)
    cols = p * 128 + c * 16 + (jj + 1)
    vals = w1[kh * 3 + kw, c] * valid.astype(f32)
    T1 = jnp.zeros((84, 256), f32).at[rows, cols].add(vals)

    c1, p1, jj1 = jnp.meshgrid(jnp.arange(6), jnp.arange(2),
                               jnp.arange(14), indexing="ij")
    t1m = jnp.zeros((1, 256), f32).at[0, p1 * 128 + c1 * 16 + jj1 + 1].set(
        jnp.broadcast_to(t1[0, c1], c1.shape))

    kh2, kw2, c2, p2, jj2 = jnp.meshgrid(jnp.arange(3), jnp.arange(3),
                                         jnp.arange(6), jnp.arange(2),
                                         jnp.arange(7), indexing="ij")
    rows2 = (kh2 * 128 + c2 * 16 + 2 * jj2 + p2 + kw2)[..., None]
    cols2 = (p2 * 512 + jj2 * 64)[..., None] + jnp.arange(64)
    vals2 = w2[(kh2 * 3 + kw2) * 6 + c2, :]
    T2 = jnp.zeros((384, 1024), f32).at[
        jnp.broadcast_to(rows2, vals2.shape),
        jnp.broadcast_to(cols2, vals2.shape)].set(vals2)

    t2m = jnp.pad(jnp.tile(t2, (1, 7)), ((0, 0), (0, 64)))
    return T1, t1m[:, 0:128].reshape(1, 1, 128), T2, t2m.reshape(1, 1, 512)


def kernel(x_nchw, w1, t1, w2, t2, wf1, t3, wf2, b2):
    n = x_nchw.shape[0]
    x = x_nchw.reshape(n, 28, 28)
    T1, t1m, T2, t2m = _build_tables(w1, t1, w2, t2)
    wf1r = jnp.pad(wf1.reshape(7, 448, 128), ((0, 0), (0, 64), (0, 0)))

    tb = _pick_tile(n, (256, 128, 64, 32, 16, 8, 4, 2, 1))
    return pl.pallas_call(
        _fused_cnn_kernel,
        out_shape=jax.ShapeDtypeStruct((n, 10), jnp.float32),
        grid=(n // tb,),
        in_specs=[
            pl.BlockSpec((tb, 28, 28), lambda i: (i, 0, 0)),
            pl.BlockSpec((84, 256), lambda i: (0, 0)),
            pl.BlockSpec((1, 1, 128), lambda i: (0, 0, 0)),
            pl.BlockSpec((384, 1024), lambda i: (0, 0)),
            pl.BlockSpec((1, 1, 512), lambda i: (0, 0, 0)),
            pl.BlockSpec((7, 512, 128), lambda i: (0, 0, 0)),
            pl.BlockSpec((1, 128), lambda i: (0, 0)),
            pl.BlockSpec((128, 10), lambda i: (0, 0)),
            pl.BlockSpec((1, 10), lambda i: (0, 0)),
        ],
        out_specs=pl.BlockSpec((tb, 10), lambda i: (i, 0)),
        scratch_shapes=[
            pltpu.VMEM((tb, 32, 28), jnp.float32),
            pltpu.VMEM((tb, 28, 128), jnp.float32),
            pltpu.VMEM((tb, 16, 128), jnp.float32),
            pltpu.VMEM((tb, 14, 512), jnp.float32),
        ],
        compiler_params=pltpu.CompilerParams(
            dimension_semantics=("parallel",),
            vmem_limit_bytes=100 * 1024 * 1024,
        ),
    )(x, T1, t1m, T2, t2m, wf1r, t3, wf2, b2)
